# baseline (speedup 1.0000x reference)
import functools

import jax
import jax.numpy as jnp
from jax import lax
from jax.experimental import pallas as pl
from jax.experimental.pallas import tpu as pltpu
from jax.experimental.pallas import tpu_sc as plsc

_B = 16384
_D = 16
_H = 32
_NROWS = 1000000
_NC, _NS = 2, 16
_NW = _NC * _NS
_BPW = _B // _NW
_G = 16
_NG = _BPW // _G

_W1UOFF = 0
_W1IOFF = _H * _D * 16
_W2OFF = 2 * _H * _D * 16
_B1OFF = _W2OFF + _H * 16
_B2OFF = _B1OFF + _H * 16
_CSTLEN = _B2OFF + 16

_sc_mesh = plsc.VectorSubcoreMesh(
    core_axis_name="c", subcore_axis_name="s", num_cores=_NC, num_subcores=_NS
)


@functools.partial(
    pl.kernel,
    out_type=jax.ShapeDtypeStruct((_B,), jnp.float32),
    mesh=_sc_mesh,
    compiler_params=pltpu.CompilerParams(needs_layout_passes=False),
    scratch_types=[
        pltpu.VMEM((_BPW,), jnp.int32),
        pltpu.VMEM((_BPW,), jnp.int32),
        pltpu.VMEM((2, _G, _D, 128), jnp.float32),
        pltpu.VMEM((_CSTLEN,), jnp.float32),
        pltpu.VMEM((_BPW,), jnp.float32),
        pltpu.SemaphoreType.DMA,
        pltpu.SemaphoreType.DMA,
    ],
)
def _sc_fused(uid_hbm, iid_hbm, utabT, itabT, cst_hbm, out_hbm,
              uids_v, iids_v, slab_v, cst_v, out_v, sem0, sem1):
    wid = lax.axis_index("s") * _NC + lax.axis_index("c")
    base = wid * _BPW
    pltpu.sync_copy(uid_hbm.at[pl.ds(base, _BPW)], uids_v)
    pltpu.sync_copy(iid_hbm.at[pl.ds(base, _BPW)], iids_v)
    pltpu.sync_copy(cst_hbm, cst_v)
    sems = (sem0, sem1)
    iota = lax.iota(jnp.int32, _G)

    def cst(off):
        return cst_v[pl.ds(off, 16)]

    def fire(tab, ids_v, g, slot):
        ids = ids_v[pl.ds(g * _G, _G)]
        cvec = (ids >> 7) * 128
        for j in range(_G):
            c = pl.multiple_of(cvec[j], 128)
            pltpu.async_copy(tab.at[:, pl.ds(c, 128)],
                             slab_v.at[slot, j], sems[slot])

    def drain(slot):
        for j in range(_G):
            pltpu.make_async_copy(utabT.at[:, pl.ds(0, 128)],
                                  slab_v.at[slot, j], sems[slot]).wait()

    def accumulate(h, slot, ids_v, g, w1off):
        ids = ids_v[pl.ds(g * _G, _G)]
        col = ids & 127
        slab = slab_v.at[slot]
        for k in range(_D):
            v = plsc.load_gather(slab, [iota, jnp.full((_G,), k, jnp.int32),
                                        col])
            for j in range(_H):
                h[j] = h[j] + v * cst(w1off + (j * _D + k) * 16)
        return h

    fire(utabT, uids_v, 0, 0)

    def body(p, _):
        fire(itabT, iids_v, p, 1)
        drain(0)
        h = [cst(_B1OFF + j * 16) for j in range(_H)]
        h = accumulate(h, 0, uids_v, p, _W1UOFF)

        @pl.when(p < _NG - 1)
        def _():
            fire(utabT, uids_v, p + 1, 0)

        drain(1)
        h = accumulate(h, 1, iids_v, p, _W1IOFF)
        acc = cst(_B2OFF)
        for j in range(_H):
            acc = acc + jnp.maximum(h[j], 0.0) * cst(_W2OFF + j * 16)
        out_v[pl.ds(p * _G, _G)] = acc
        return 0

    lax.fori_loop(0, _NG, body, 0)
    pltpu.sync_copy(out_v, out_hbm.at[pl.ds(base, _BPW)])


def kernel(user_ids, item_ids, user_table, item_table, W1, b1, W2, b2):
    user_ids = user_ids.astype(jnp.int32)
    item_ids = item_ids.astype(jnp.int32)
    w1u = jnp.broadcast_to(W1[:, :_D][:, :, None], (_H, _D, 16)).reshape(-1)
    w1i = jnp.broadcast_to(W1[:, _D:][:, :, None], (_H, _D, 16)).reshape(-1)
    w2b = jnp.broadcast_to(W2.reshape(_H)[:, None], (_H, 16)).reshape(-1)
    b1b = jnp.broadcast_to(b1[:, None], (_H, 16)).reshape(-1)
    b2b = jnp.broadcast_to(b2.reshape(1, 1), (1, 16)).reshape(-1)
    cst = jnp.concatenate([w1u, w1i, w2b, b1b, b2b])
    out_flat = _sc_fused(user_ids, item_ids, user_table.T, item_table.T, cst)
    return out_flat.reshape(_B, 1)


# speedup vs baseline: 1.1632x; 1.1632x over previous
import functools

import jax
import jax.numpy as jnp
from jax import lax
from jax.experimental import pallas as pl
from jax.experimental.pallas import tpu as pltpu
from jax.experimental.pallas import tpu_sc as plsc

_B = 16384
_D = 16
_H = 32
_NROWS = 1000000
_NC, _NS = 2, 16
_NW = _NC * _NS
_BPW = _B // _NW
_G = 16
_NG = _BPW // _G

_W1UOFF = 0
_W1IOFF = _H * _D * 16
_W2OFF = 2 * _H * _D * 16
_B1OFF = _W2OFF + _H * 16
_B2OFF = _B1OFF + _H * 16
_CSTLEN = _B2OFF + 16

_sc_mesh = plsc.VectorSubcoreMesh(
    core_axis_name="c", subcore_axis_name="s", num_cores=_NC, num_subcores=_NS
)


@functools.partial(
    pl.kernel,
    out_type=jax.ShapeDtypeStruct((_B,), jnp.float32),
    mesh=_sc_mesh,
    compiler_params=pltpu.CompilerParams(needs_layout_passes=False),
    scratch_types=[
        pltpu.VMEM((_BPW,), jnp.int32),
        pltpu.VMEM((_BPW,), jnp.int32),
        pltpu.VMEM((2, _G, _D, 128), jnp.float32),
        pltpu.VMEM((_CSTLEN,), jnp.float32),
        pltpu.VMEM((_NG * _H * 16,), jnp.float32),
        pltpu.VMEM((_BPW,), jnp.float32),
        pltpu.SemaphoreType.DMA,
        pltpu.SemaphoreType.DMA,
    ],
)
def _sc_fused(uid_hbm, iid_hbm, utabT, itabT, cst_hbm, out_hbm,
              uids_v, iids_v, slab_v, cst_v, hbuf_v, out_v, sem0, sem1):
    wid = lax.axis_index("s") * _NC + lax.axis_index("c")
    base = wid * _BPW
    pltpu.sync_copy(uid_hbm.at[pl.ds(base, _BPW)], uids_v)
    pltpu.sync_copy(iid_hbm.at[pl.ds(base, _BPW)], iids_v)
    pltpu.sync_copy(cst_hbm, cst_v)
    sems = (sem0, sem1)
    iota = lax.iota(jnp.int32, _G)

    def cst(off):
        return cst_v[pl.ds(off, 16)]

    def fire(tab, ids_v, g, slot):
        ids = ids_v[pl.ds(g * _G, _G)]
        cvec = (ids >> 7) * 128
        for j in range(_G):
            c = pl.multiple_of(cvec[j], 128)
            pltpu.async_copy(tab.at[:, pl.ds(c, 128)],
                             slab_v.at[slot, j], sems[slot])

    def drain(slot):
        for j in range(_G):
            pltpu.make_async_copy(utabT.at[:, pl.ds(0, 128)],
                                  slab_v.at[slot, j], sems[slot]).wait()

    def accumulate(h, slot, ids_v, g, w1off):
        ids = ids_v[pl.ds(g * _G, _G)]
        col = ids & 127
        slab = slab_v.at[slot]
        for k in range(_D):
            v = plsc.load_gather(slab, [iota, jnp.full((_G,), k, jnp.int32),
                                        col])
            for j in range(_H):
                h[j] = h[j] + v * cst(w1off + (j * _D + k) * 16)
        return h

    def user_step(g, slot):
        drain(slot)
        h = [cst(_B1OFF + j * 16) for j in range(_H)]
        h = accumulate(h, slot, uids_v, g, _W1UOFF)
        for j in range(_H):
            hbuf_v[pl.ds((g * _H + j) * 16, 16)] = h[j]

    def item_step(g, slot):
        drain(slot)
        h = [hbuf_v[pl.ds((g * _H + j) * 16, 16)] for j in range(_H)]
        h = accumulate(h, slot, iids_v, g, _W1IOFF)
        acc = cst(_B2OFF)
        for j in range(_H):
            acc = acc + jnp.maximum(h[j], 0.0) * cst(_W2OFF + j * 16)
        out_v[pl.ds(g * _G, _G)] = acc

    def run_table(tab, ids_v, step):
        fire(tab, ids_v, 0, 0)
        fire(tab, ids_v, 1, 1)

        def body(p, _):
            g0 = 2 * p
            step(g0, 0)

            @pl.when(g0 + 2 < _NG)
            def _():
                fire(tab, ids_v, g0 + 2, 0)

            step(g0 + 1, 1)

            @pl.when(g0 + 3 < _NG)
            def _():
                fire(tab, ids_v, g0 + 3, 1)

            return 0

        lax.fori_loop(0, _NG // 2, body, 0)

    run_table(utabT, uids_v, user_step)
    run_table(itabT, iids_v, item_step)
    pltpu.sync_copy(out_v, out_hbm.at[pl.ds(base, _BPW)])


def kernel(user_ids, item_ids, user_table, item_table, W1, b1, W2, b2):
    user_ids = user_ids.astype(jnp.int32)
    item_ids = item_ids.astype(jnp.int32)
    w1u = jnp.broadcast_to(W1[:, :_D][:, :, None], (_H, _D, 16)).reshape(-1)
    w1i = jnp.broadcast_to(W1[:, _D:][:, :, None], (_H, _D, 16)).reshape(-1)
    w2b = jnp.broadcast_to(W2.reshape(_H)[:, None], (_H, 16)).reshape(-1)
    b1b = jnp.broadcast_to(b1[:, None], (_H, 16)).reshape(-1)
    b2b = jnp.broadcast_to(b2.reshape(1, 1), (1, 16)).reshape(-1)
    cst = jnp.concatenate([w1u, w1i, w2b, b1b, b2b])
    out_flat = _sc_fused(user_ids, item_ids, user_table.T, item_table.T, cst)
    return out_flat.reshape(_B, 1)


# speedup vs baseline: 1.4354x; 1.2340x over previous
import functools

import jax
import jax.numpy as jnp
from jax import lax
from jax.experimental import pallas as pl
from jax.experimental.pallas import tpu as pltpu
from jax.experimental.pallas import tpu_sc as plsc

_B = 16384
_D = 16
_H = 32
_NROWS = 1000000
_NC, _NS = 2, 16
_NW = _NC * _NS
_BPW = _B // _NW
_G = 16
_NG = _BPW // _G

_sc_mesh = plsc.VectorSubcoreMesh(
    core_axis_name="c", subcore_axis_name="s", num_cores=_NC, num_subcores=_NS
)


@functools.partial(
    pl.kernel,
    out_type=(
        jax.ShapeDtypeStruct((_D * _B,), jnp.float32),
        jax.ShapeDtypeStruct((_D * _B,), jnp.float32),
    ),
    mesh=_sc_mesh,
    compiler_params=pltpu.CompilerParams(needs_layout_passes=False),
    scratch_types=[
        pltpu.VMEM((_BPW,), jnp.int32),
        pltpu.VMEM((_BPW,), jnp.int32),
        pltpu.VMEM((2, _G, _D, 128), jnp.float32),
        pltpu.VMEM((_D * _BPW,), jnp.float32),
        pltpu.VMEM((_D * _BPW,), jnp.float32),
        pltpu.SemaphoreType.DMA,
        pltpu.SemaphoreType.DMA,
        pltpu.SemaphoreType.DMA,
    ],
)
def _sc_gather(uid_hbm, iid_hbm, utabT, itabT, u_out, i_out,
               uids_v, iids_v, slab_v, uout_v, iout_v, sem0, sem1, osem):
    wid = lax.axis_index("s") * _NC + lax.axis_index("c")
    base = wid * _BPW
    pltpu.sync_copy(uid_hbm.at[pl.ds(base, _BPW)], uids_v)
    pltpu.sync_copy(iid_hbm.at[pl.ds(base, _BPW)], iids_v)
    sems = (sem0, sem1)
    iota = lax.iota(jnp.int32, _G)

    def fire(g, slot, tab, ids_v):
        ids = ids_v[pl.ds(g * _G, _G)]
        cvec = (ids >> 7) * 128
        for j in range(_G):
            c = pl.multiple_of(cvec[j], 128)
            pltpu.async_copy(tab.at[:, pl.ds(c, 128)],
                             slab_v.at[slot, j], sems[slot])

    def drain(slot, tab):
        for j in range(_G):
            pltpu.make_async_copy(tab.at[:, pl.ds(0, 128)],
                                  slab_v.at[slot, j], sems[slot]).wait()

    def extract(g, slot, ids_v, out_v):
        ids = ids_v[pl.ds(g * _G, _G)]
        col = ids & 127
        s_idx = g * _G + iota
        slab = slab_v.at[slot]
        for k in range(_D):
            v = plsc.load_gather(slab, [iota, jnp.full((_G,), k, jnp.int32),
                                        col])
            plsc.store_scatter(out_v, [k * _BPW + s_idx], v)

    def run_table(tab, ids_v, out_v):
        fire(0, 0, tab, ids_v)

        def body(p, _):
            g0 = 2 * p
            fire(g0 + 1, 1, tab, ids_v)
            drain(0, tab)
            extract(g0, 0, ids_v, out_v)

            @pl.when(p < _NG // 2 - 1)
            def _():
                fire(g0 + 2, 0, tab, ids_v)

            drain(1, tab)
            extract(g0 + 1, 1, ids_v, out_v)
            return 0

        lax.fori_loop(0, _NG // 2, body, 0)

    run_table(utabT, uids_v, uout_v)
    run_table(itabT, iids_v, iout_v)
    for k in range(_D):
        pltpu.async_copy(uout_v.at[pl.ds(k * _BPW, _BPW)],
                         u_out.at[pl.ds(k * _B + base, _BPW)], osem)
        pltpu.async_copy(iout_v.at[pl.ds(k * _BPW, _BPW)],
                         i_out.at[pl.ds(k * _B + base, _BPW)], osem)
    for k in range(_D):
        pltpu.make_async_copy(uout_v.at[pl.ds(k * _BPW, _BPW)],
                              u_out.at[pl.ds(k * _B + base, _BPW)],
                              osem).wait()
        pltpu.make_async_copy(iout_v.at[pl.ds(k * _BPW, _BPW)],
                              i_out.at[pl.ds(k * _B + base, _BPW)],
                              osem).wait()


_BLK = 4096


def _mlp_body(u_ref, i_ref, w1u_ref, w1i_ref, b1_ref, w2_ref, b2_ref,
              out_ref):
    ones = jnp.full((1, _BLK), 1.0, jnp.float32)
    h = lax.dot_general(w1u_ref[...], u_ref[...], (((1,), (0,)), ((), ())),
                        preferred_element_type=jnp.float32)
    h += lax.dot_general(w1i_ref[...], i_ref[...], (((1,), (0,)), ((), ())),
                         preferred_element_type=jnp.float32)
    h += lax.dot_general(b1_ref[...], ones, (((1,), (0,)), ((), ())),
                         preferred_element_type=jnp.float32)
    h = jnp.maximum(h, 0.0)
    out = lax.dot_general(w2_ref[...], h, (((1,), (0,)), ((), ())),
                          preferred_element_type=jnp.float32)
    out += lax.dot_general(b2_ref[...], ones, (((1,), (0,)), ((), ())),
                           preferred_element_type=jnp.float32)
    out_ref[...] = out


def _mlp(uT, iT, w1u, w1i, b1, W2, b2):
    return pl.pallas_call(
        _mlp_body,
        grid=(_B // _BLK,),
        in_specs=[
            pl.BlockSpec((_D, _BLK), lambda b: (0, b)),
            pl.BlockSpec((_D, _BLK), lambda b: (0, b)),
            pl.BlockSpec((_H, _D), lambda b: (0, 0)),
            pl.BlockSpec((_H, _D), lambda b: (0, 0)),
            pl.BlockSpec((_H, 1), lambda b: (0, 0)),
            pl.BlockSpec((1, _H), lambda b: (0, 0)),
            pl.BlockSpec((1, 1), lambda b: (0, 0)),
        ],
        out_specs=pl.BlockSpec((1, _BLK), lambda b: (0, b)),
        out_shape=jax.ShapeDtypeStruct((1, _B), jnp.float32),
    )(uT, iT, w1u, w1i, b1, W2, b2)


def kernel(user_ids, item_ids, user_table, item_table, W1, b1, W2, b2):
    user_ids = user_ids.astype(jnp.int32)
    item_ids = item_ids.astype(jnp.int32)
    u_flat, i_flat = _sc_gather(user_ids, item_ids,
                                user_table.T, item_table.T)
    uT = u_flat.reshape(_D, _B)
    iT = i_flat.reshape(_D, _B)
    outT = _mlp(uT, iT, W1[:, :_D], W1[:, _D:], b1.reshape(_H, 1),
                W2, b2.reshape(1, 1))
    return outT.reshape(_B, 1)


# speedup vs baseline: 1.4810x; 1.0318x over previous
import functools

import jax
import jax.numpy as jnp
from jax import lax
from jax.experimental import pallas as pl
from jax.experimental.pallas import tpu as pltpu
from jax.experimental.pallas import tpu_sc as plsc

_B = 16384
_D = 16
_H = 32
_NROWS = 1000000
_NC, _NS = 2, 16
_NW = _NC * _NS
_BPW = _B // _NW
_G = 16
_NG = _BPW // _G

_sc_mesh = plsc.VectorSubcoreMesh(
    core_axis_name="c", subcore_axis_name="s", num_cores=_NC, num_subcores=_NS
)


@functools.partial(
    pl.kernel,
    out_type=(
        jax.ShapeDtypeStruct((_D * _B,), jnp.float32),
        jax.ShapeDtypeStruct((_D * _B,), jnp.float32),
    ),
    mesh=_sc_mesh,
    compiler_params=pltpu.CompilerParams(needs_layout_passes=False),
    scratch_types=[
        pltpu.VMEM((_BPW,), jnp.int32),
        pltpu.VMEM((_BPW,), jnp.int32),
        pltpu.VMEM((3, _G, _D, 128), jnp.float32),
        pltpu.VMEM((_D * _BPW,), jnp.float32),
        pltpu.VMEM((_D * _BPW,), jnp.float32),
        pltpu.SemaphoreType.DMA,
        pltpu.SemaphoreType.DMA,
        pltpu.SemaphoreType.DMA,
        pltpu.SemaphoreType.DMA,
    ],
)
def _sc_gather(uid_hbm, iid_hbm, utabT, itabT, u_out, i_out,
               uids_v, iids_v, slab_v, uout_v, iout_v, sem0, sem1, sem2,
               osem):
    wid = lax.axis_index("s") * _NC + lax.axis_index("c")
    base = wid * _BPW
    pltpu.sync_copy(uid_hbm.at[pl.ds(base, _BPW)], uids_v)
    pltpu.sync_copy(iid_hbm.at[pl.ds(base, _BPW)], iids_v)
    sems = (sem0, sem1, sem2)
    iota = lax.iota(jnp.int32, _G)

    def fire(g, slot, tab, ids_v):
        ids = ids_v[pl.ds(g * _G, _G)]
        cvec = (ids >> 7) * 128
        for j in range(_G):
            c = pl.multiple_of(cvec[j], 128)
            pltpu.async_copy(tab.at[:, pl.ds(c, 128)],
                             slab_v.at[slot, j], sems[slot])

    def drain(slot, tab):
        for j in range(_G):
            pltpu.make_async_copy(tab.at[:, pl.ds(0, 128)],
                                  slab_v.at[slot, j], sems[slot]).wait()

    def extract(g, slot, ids_v, out_v):
        ids = ids_v[pl.ds(g * _G, _G)]
        col = ids & 127
        s_idx = g * _G + iota
        slab = slab_v.at[slot]
        for k in range(_D):
            v = plsc.load_gather(slab, [iota, jnp.full((_G,), k, jnp.int32),
                                        col])
            plsc.store_scatter(out_v, [k * _BPW + s_idx], v)

    def run_table(tab, ids_v, out_v):
        for g in range(3):
            fire(g, g % 3, tab, ids_v)

        def body(q, _):
            g0 = 6 * q
            for m in range(6):
                g = g0 + m
                slot = m % 3
                drain(slot, tab)
                extract(g, slot, ids_v, out_v)

                @pl.when(g + 3 < _NG)
                def _():
                    fire(g + 3, slot, tab, ids_v)

            return 0

        lax.fori_loop(0, _NG // 6, body, 0)
        for g in range(_NG - _NG % 6, _NG):
            drain(g % 3, tab)
            extract(g, g % 3, ids_v, out_v)

    run_table(utabT, uids_v, uout_v)
    run_table(itabT, iids_v, iout_v)
    for k in range(_D):
        pltpu.async_copy(uout_v.at[pl.ds(k * _BPW, _BPW)],
                         u_out.at[pl.ds(k * _B + base, _BPW)], osem)
        pltpu.async_copy(iout_v.at[pl.ds(k * _BPW, _BPW)],
                         i_out.at[pl.ds(k * _B + base, _BPW)], osem)
    for k in range(_D):
        pltpu.make_async_copy(uout_v.at[pl.ds(k * _BPW, _BPW)],
                              u_out.at[pl.ds(k * _B + base, _BPW)],
                              osem).wait()
        pltpu.make_async_copy(iout_v.at[pl.ds(k * _BPW, _BPW)],
                              i_out.at[pl.ds(k * _B + base, _BPW)],
                              osem).wait()


_BLK = 4096


def _mlp_body(u_ref, i_ref, w1u_ref, w1i_ref, b1_ref, w2_ref, b2_ref,
              out_ref):
    ones = jnp.full((1, _BLK), 1.0, jnp.float32)
    h = lax.dot_general(w1u_ref[...], u_ref[...], (((1,), (0,)), ((), ())),
                        preferred_element_type=jnp.float32)
    h += lax.dot_general(w1i_ref[...], i_ref[...], (((1,), (0,)), ((), ())),
                         preferred_element_type=jnp.float32)
    h += lax.dot_general(b1_ref[...], ones, (((1,), (0,)), ((), ())),
                         preferred_element_type=jnp.float32)
    h = jnp.maximum(h, 0.0)
    out = lax.dot_general(w2_ref[...], h, (((1,), (0,)), ((), ())),
                          preferred_element_type=jnp.float32)
    out += lax.dot_general(b2_ref[...], ones, (((1,), (0,)), ((), ())),
                           preferred_element_type=jnp.float32)
    out_ref[...] = out


def _mlp(uT, iT, w1u, w1i, b1, W2, b2):
    return pl.pallas_call(
        _mlp_body,
        grid=(_B // _BLK,),
        in_specs=[
            pl.BlockSpec((_D, _BLK), lambda b: (0, b)),
            pl.BlockSpec((_D, _BLK), lambda b: (0, b)),
            pl.BlockSpec((_H, _D), lambda b: (0, 0)),
            pl.BlockSpec((_H, _D), lambda b: (0, 0)),
            pl.BlockSpec((_H, 1), lambda b: (0, 0)),
            pl.BlockSpec((1, _H), lambda b: (0, 0)),
            pl.BlockSpec((1, 1), lambda b: (0, 0)),
        ],
        out_specs=pl.BlockSpec((1, _BLK), lambda b: (0, b)),
        out_shape=jax.ShapeDtypeStruct((1, _B), jnp.float32),
    )(uT, iT, w1u, w1i, b1, W2, b2)


def kernel(user_ids, item_ids, user_table, item_table, W1, b1, W2, b2):
    user_ids = user_ids.astype(jnp.int32)
    item_ids = item_ids.astype(jnp.int32)
    u_flat, i_flat = _sc_gather(user_ids, item_ids,
                                user_table.T, item_table.T)
    uT = u_flat.reshape(_D, _B)
    iT = i_flat.reshape(_D, _B)
    outT = _mlp(uT, iT, W1[:, :_D], W1[:, _D:], b1.reshape(_H, 1),
                W2, b2.reshape(1, 1))
    return outT.reshape(_B, 1)
